# baseline (speedup 1.0000x reference)
import jax
import jax.numpy as jnp
from jax.experimental import pallas as pl
from jax.experimental.pallas import tpu as pltpu


_BLOCK_N = 2000


def _copy(feat_hbm, buf, sems, slot, blk, B):
    return pltpu.make_async_copy(
        feat_hbm.at[pl.ds(blk * B, B)],
        buf.at[slot],
        sems.at[slot],
    )


def _sumdis_kernel(dist_ref, feat_hbm, out_ref, buf, sems):
    i = pl.program_id(0)
    ni = pl.num_programs(0)
    B = out_ref.shape[0]
    slot = jax.lax.rem(i, 2)
    nslot = jax.lax.rem(i + 1, 2)

    @pl.when(i == 0)
    def _():
        _copy(feat_hbm, buf, sems, 0, 0, B).start()

    @pl.when(i + 1 < ni)
    def _():
        _copy(feat_hbm, buf, sems, nslot, i + 1, B).start()

    _copy(feat_hbm, buf, sems, slot, i, B).wait()

    d = dist_ref[...]
    norm = jnp.sum(d, axis=1, keepdims=True)
    w = d / norm
    f = buf[slot]
    z = (w[:, 0:1] * f[:, 0, :]
         + w[:, 1:2] * f[:, 1, :]
         + w[:, 2:3] * f[:, 2, :])
    out_ref[...] = z


def kernel(distance, interpolated_feature):
    N, K = distance.shape
    D = interpolated_feature.shape[-1]
    B = _BLOCK_N
    grid = (N // B,)
    return pl.pallas_call(
        _sumdis_kernel,
        grid=grid,
        in_specs=[
            pl.BlockSpec((B, K), lambda i: (i, 0)),
            pl.BlockSpec(memory_space=pltpu.MemorySpace.HBM),
        ],
        out_specs=pl.BlockSpec((B, D), lambda i: (i, 0)),
        out_shape=jax.ShapeDtypeStruct((N, D), interpolated_feature.dtype),
        scratch_shapes=[
            pltpu.MemorySpace.VMEM((2, B, K, D), jnp.float32),
            pltpu.SemaphoreType.DMA((2,)),
        ],
    )(distance, interpolated_feature)


# speedup vs baseline: 1.2297x; 1.2297x over previous
import jax
import jax.numpy as jnp
from jax.experimental import pallas as pl
from jax.experimental.pallas import tpu as pltpu


_BLOCK_N = 2000


def _sumdis_kernel(dist_ref, feat_ref, out_ref):
    d = dist_ref[...]
    norm = jnp.sum(d, axis=1, keepdims=True)
    w = d / norm
    f = feat_ref[...]
    D = out_ref.shape[-1]
    z = (w[:, 0:1] * f[:, 0:D]
         + w[:, 1:2] * f[:, D:2 * D]
         + w[:, 2:3] * f[:, 2 * D:3 * D])
    out_ref[...] = z


def kernel(distance, interpolated_feature):
    N, K = distance.shape
    D = interpolated_feature.shape[-1]
    feat2 = interpolated_feature.reshape(N, K * D)
    B = _BLOCK_N
    grid = (N // B,)
    return pl.pallas_call(
        _sumdis_kernel,
        grid=grid,
        in_specs=[
            pl.BlockSpec((B, K), lambda i: (i, 0)),
            pl.BlockSpec((B, K * D), lambda i: (i, 0)),
        ],
        out_specs=pl.BlockSpec((B, D), lambda i: (i, 0)),
        out_shape=jax.ShapeDtypeStruct((N, D), interpolated_feature.dtype),
        compiler_params=pltpu.CompilerParams(
            allow_input_fusion=(False, True),
        ),
    )(distance, feat2)


# speedup vs baseline: 5.9916x; 4.8724x over previous
import jax
import jax.numpy as jnp
from jax.experimental import pallas as pl


_BLOCK_N = 2048


def _sumdis_kernel(dist_ref, f0_ref, f1_ref, f2_ref, out_ref):
    i = pl.program_id(0)
    B = out_ref.shape[0]
    d = dist_ref[:, pl.ds(i * B, B)]
    norm = jnp.sum(d, axis=0, keepdims=True)
    w3 = d / norm
    w = jnp.transpose(w3)
    z = (w[:, 0:1] * f0_ref[0]
         + w[:, 1:2] * f1_ref[0]
         + w[:, 2:3] * f2_ref[0])
    out_ref[...] = z


def kernel(distance, interpolated_feature):
    N, K = distance.shape
    D = interpolated_feature.shape[-1]
    feat_t = jnp.transpose(interpolated_feature, (1, 0, 2))
    dist_t = jnp.transpose(distance)
    B = _BLOCK_N
    num_blocks = pl.cdiv(N, B)
    pad = num_blocks * B - N
    dist_tp = jnp.pad(dist_t, ((0, 0), (0, pad)), constant_values=1.0)
    grid = (num_blocks,)
    return pl.pallas_call(
        _sumdis_kernel,
        grid=grid,
        in_specs=[
            pl.BlockSpec((K, num_blocks * B), lambda i: (0, 0)),
            pl.BlockSpec((1, B, D), lambda i: (0, i, 0)),
            pl.BlockSpec((1, B, D), lambda i: (1, i, 0)),
            pl.BlockSpec((1, B, D), lambda i: (2, i, 0)),
        ],
        out_specs=pl.BlockSpec((B, D), lambda i: (i, 0)),
        out_shape=jax.ShapeDtypeStruct((N, D), interpolated_feature.dtype),
    )(dist_tp, feat_t, feat_t, feat_t)


# speedup vs baseline: 6.3382x; 1.0579x over previous
import jax
import jax.numpy as jnp
from jax.experimental import pallas as pl


_BLOCK_N = 4096


def _sumdis_kernel(dist_ref, f0_ref, f1_ref, f2_ref, out_ref):
    d = dist_ref[...]
    norm = jnp.sum(d, axis=0, keepdims=True)
    w3 = d / norm
    w = jnp.transpose(w3)
    z = (w[:, 0:1] * f0_ref[0]
         + w[:, 1:2] * f1_ref[0]
         + w[:, 2:3] * f2_ref[0])
    out_ref[...] = z


def kernel(distance, interpolated_feature):
    N, K = distance.shape
    D = interpolated_feature.shape[-1]
    feat_t = jnp.transpose(interpolated_feature, (1, 0, 2))
    dist_t = jnp.transpose(distance)
    B = _BLOCK_N
    grid = (pl.cdiv(N, B),)
    return pl.pallas_call(
        _sumdis_kernel,
        grid=grid,
        in_specs=[
            pl.BlockSpec((K, B), lambda i: (0, i)),
            pl.BlockSpec((1, B, D), lambda i: (0, i, 0)),
            pl.BlockSpec((1, B, D), lambda i: (1, i, 0)),
            pl.BlockSpec((1, B, D), lambda i: (2, i, 0)),
        ],
        out_specs=pl.BlockSpec((B, D), lambda i: (i, 0)),
        out_shape=jax.ShapeDtypeStruct((N, D), interpolated_feature.dtype),
    )(dist_t, feat_t, feat_t, feat_t)


# speedup vs baseline: 6.3831x; 1.0071x over previous
import jax
import jax.numpy as jnp
from jax.experimental import pallas as pl


_BLOCK_N = 5120


def _sumdis_kernel(dist_ref, f0_ref, f1_ref, f2_ref, out_ref):
    d = dist_ref[...]
    norm = jnp.sum(d, axis=0, keepdims=True)
    w3 = d / norm
    w = jnp.transpose(w3)
    z = (w[:, 0:1] * f0_ref[0]
         + w[:, 1:2] * f1_ref[0]
         + w[:, 2:3] * f2_ref[0])
    out_ref[...] = z


def kernel(distance, interpolated_feature):
    N, K = distance.shape
    D = interpolated_feature.shape[-1]
    feat_t = jnp.transpose(interpolated_feature, (1, 0, 2))
    dist_t = jnp.transpose(distance)
    B = _BLOCK_N
    grid = (pl.cdiv(N, B),)
    return pl.pallas_call(
        _sumdis_kernel,
        grid=grid,
        in_specs=[
            pl.BlockSpec((K, B), lambda i: (0, i)),
            pl.BlockSpec((1, B, D), lambda i: (0, i, 0)),
            pl.BlockSpec((1, B, D), lambda i: (1, i, 0)),
            pl.BlockSpec((1, B, D), lambda i: (2, i, 0)),
        ],
        out_specs=pl.BlockSpec((B, D), lambda i: (i, 0)),
        out_shape=jax.ShapeDtypeStruct((N, D), interpolated_feature.dtype),
    )(dist_t, feat_t, feat_t, feat_t)
